# baseline (speedup 1.0000x reference)
import functools

import jax
import jax.numpy as jnp
from jax import lax
from jax.experimental import pallas as pl
from jax.experimental.pallas import tpu as pltpu
from jax.experimental.pallas import tpu_sc as plsc

N = 10000
E = 160000
D = 256
H = 128
NC, NS = 2, 16
NP = 10240
RPT = NP // NS
GK = 128
GNIT = RPT // GK
EPT = E // NS
EPTP = 10240
K = 64
NIT = EPTP // K
NBLK = 10
NBC = NIT // NBLK
VCH = EPTP // 16
RB = 1024
NRB = NP // RB

f32 = jnp.float32
i32 = jnp.int32

_mesh = plsc.VectorSubcoreMesh(core_axis_name="c", subcore_axis_name="s")


@functools.partial(
    pl.kernel,
    out_type=(
        jax.ShapeDtypeStruct((NC, NP, D), f32),
        jax.ShapeDtypeStruct((NC, NS, NP), f32),
    ),
    mesh=_mesh,
    compiler_params=pltpu.CompilerParams(needs_layout_passes=False),
    scratch_types=(
        pltpu.VMEM((GNIT, GK), i32),
        pltpu.VMEM((GK, D), f32),
        pltpu.VMEM((GK, D), f32),
        pltpu.VMEM((GK, D), f32),
        pltpu.VMEM((EPTP,), i32),
        pltpu.VMEM((NP,), f32),
        pltpu.SemaphoreType.DMA,
        pltpu.SemaphoreType.DMA,
        pltpu.SemaphoreType.DMA,
        pltpu.SemaphoreType.DMA,
        pltpu.SemaphoreType.DMA,
        pltpu.SemaphoreType.DMA,
    ),
)
def _p1(ids_r, dst_r, emb_r, zc_r, h0_o, deg_o, ids_v, r0_v, r1_v, r2_v,
        dst_v, acc_v, sg0, sg1, sg2, st0, st1, st2):
    c = lax.axis_index("c")
    s = lax.axis_index("s")
    B = (r0_v, r1_v, r2_v)
    SG = (sg0, sg1, sg2)
    ST = (st0, st1, st2)
    dummy = emb_r.at[pl.ds(0, GK), :]

    def iG(b, j):
        pltpu.async_copy(emb_r.at[ids_v.at[j]], B[b], SG[b])

    def wG(b):
        pltpu.make_async_copy(dummy, B[b], SG[b]).wait()

    def iSt(b, j):
        pltpu.async_copy(B[b], h0_o.at[c, pl.ds(s * RPT + j * GK, GK), :],
                         ST[b])

    def wSt(b):
        pltpu.make_async_copy(dummy, B[b], ST[b]).wait()

    pltpu.sync_copy(ids_r.at[c, s], ids_v)
    for j in range(3):
        iG(j, j)
    pltpu.sync_copy(dst_r.at[c, s], dst_v)
    pltpu.sync_copy(zc_r, acc_v)
    ones = jnp.full((16,), 1.0, f32)

    def deg_body(k, carry):
        for t in range(4):
            idx = dst_v[pl.ds((k * 4 + t) * 16, 16)]
            plsc.addupdate_scatter(acc_v, [idx], ones)
        return carry

    lax.fori_loop(0, VCH // 4, deg_body, 0)
    pltpu.sync_copy(acc_v, deg_o.at[c, s])
    for j in range(GNIT):
        b = j % 3
        if j >= 3:
            wSt(b)
            iG(b, j)
        wG(b)
        iSt(b, j)
    for j in range(GNIT - 3, GNIT):
        wSt(j % 3)


def _p2a_body(dp_ref, norm_ref):
    d = jnp.sum(dp_ref[...], axis=1) + 1.0
    norm_ref[...] = lax.rsqrt(d)


_p2a = pl.pallas_call(
    _p2a_body,
    out_shape=jax.ShapeDtypeStruct((NC, NP), f32),
)


def _p2b_body(h0_ref, n_ref, g_ref):
    ncol = n_ref[0, 0]
    gf = h0_ref[0] * ncol
    g_ref[0, 0] = gf[:, :H]
    g_ref[1, 0] = gf[:, H:]


_p2b = pl.pallas_call(
    _p2b_body,
    grid=(NC, NRB),
    in_specs=[
        pl.BlockSpec((1, RB, D), lambda g, b: (g, b, 0)),
        pl.BlockSpec((1, 1, RB, 1), lambda g, b: (g, b, 0, 0)),
    ],
    out_specs=pl.BlockSpec((2, 1, RB, H), lambda g, b: (0, g, b, 0)),
    out_shape=jax.ShapeDtypeStruct((2, NC, NP, H), f32),
)


@functools.partial(
    pl.kernel,
    out_type=jax.ShapeDtypeStruct((NC, 2, NP, H), f32),
    mesh=_mesh,
    compiler_params=pltpu.CompilerParams(needs_layout_passes=False),
    scratch_types=(
        pltpu.VMEM((2, NBC, K), i32),
        pltpu.VMEM((2, NBC, K), i32),
        pltpu.VMEM((K, H), f32),
        pltpu.VMEM((K, H), f32),
        pltpu.VMEM((K, H), f32),
        pltpu.VMEM((K, H), f32),
        pltpu.VMEM_SHARED((NP, H), f32),
        pltpu.SemaphoreType.DMA,
        pltpu.SemaphoreType.DMA,
        pltpu.SemaphoreType.DMA,
        pltpu.SemaphoreType.DMA,
        pltpu.SemaphoreType.DMA,
        pltpu.SemaphoreType.DMA,
        pltpu.SemaphoreType.DMA,
        pltpu.SemaphoreType.DMA,
        pltpu.SemaphoreType.DMA,
    ),
)
def _p3(srcA_r, dst3_r, g2_r, zrows_r, agg_o, si_v, di_v, r0_v, r1_v, r2_v,
        r3_v, acc_sh, sg0, sg1, sg2, sg3, ss0, ss1, ss2, ss3, sx):
    c = lax.axis_index("c")
    s = lax.axis_index("s")
    B = (r0_v, r1_v, r2_v, r3_v)
    SG = (sg0, sg1, sg2, sg3)
    SS = (ss0, ss1, ss2, ss3)
    dummy = g2_r.at[pl.ds(0, K), :]
    dummy_i = srcA_r.at[0, 0, 0, 0]

    def wG(b):
        pltpu.make_async_copy(dummy, B[b], SG[b]).wait()

    def wS(b):
        pltpu.make_async_copy(dummy, B[b], SS[b]).wait()

    def stage(g, blk, p):
        pltpu.async_copy(srcA_r.at[c, g, s, blk], si_v.at[p], sx)
        pltpu.async_copy(dst3_r.at[g, s, blk], di_v.at[p], sx)

    def wstage(p):
        pltpu.make_async_copy(dummy_i, si_v.at[p], sx).wait()
        pltpu.make_async_copy(dummy_i, di_v.at[p], sx).wait()

    stage(0, 0, 0)
    for g in range(2):
        pltpu.sync_copy(zrows_r, r0_v)
        for j in range(RPT // K):
            pltpu.sync_copy(r0_v, acc_sh.at[pl.ds(s * RPT + j * K, K), :])
        plsc.subcore_barrier()

        for blk in range(NBLK):
            p = blk % 2
            wstage(p)
            if blk + 1 < NBLK:
                stage(g, blk + 1, 1 - p)
            elif g == 0:
                stage(1, 0, 1 - p)
            si = si_v.at[p]
            di = di_v.at[p]

            def iG(b, j):
                pltpu.async_copy(g2_r.at[si.at[j]], B[b], SG[b])

            def iS(b, j):
                pltpu.async_copy(B[b], acc_sh.at[di.at[j]], SS[b],
                                 add=True)

            iG(0, 0)
            iG(1, 1)
            wG(0); iS(0, 0); iG(2, 2)
            wG(1); iS(1, 1); iG(3, 3)

            def body(ii, carry):
                j0 = 2 + ii * 4
                for t in range(4):
                    j = j0 + t
                    b = (2 + t) % 4
                    wG(b); iS(b, j); wS(t); iG(t, j + 2)
                return carry

            lax.fori_loop(0, 3, body, 0)
            wG(2); iS(2, 14); wS(0)
            wG(3); iS(3, 15); wS(1)
            wS(2); wS(3)

        plsc.subcore_barrier()
        for j in range(RPT // K):
            r0 = s * RPT + j * K
            pltpu.sync_copy(acc_sh.at[pl.ds(r0, K), :], r0_v)
            pltpu.sync_copy(r0_v, agg_o.at[c, g, pl.ds(r0, K), :])
        plsc.subcore_barrier()


@functools.partial(
    pl.kernel,
    out_type=jax.ShapeDtypeStruct((NC, NS, NP), f32),
    mesh=_mesh,
    compiler_params=pltpu.CompilerParams(needs_layout_passes=False),
    scratch_types=(
        pltpu.VMEM((EPTP,), i32),
        pltpu.VMEM((EPTP,), i32),
        pltpu.VMEM((NP,), f32),
        pltpu.VMEM((NP,), f32),
    ),
)
def _p3c(srcf_r, dstf_r, norm_r, zc_r, cp_o, sf_v, df_v, nrm_v, cacc_v):
    c = lax.axis_index("c")
    s = lax.axis_index("s")
    pltpu.sync_copy(norm_r.at[c], nrm_v)
    pltpu.sync_copy(srcf_r.at[c, s], sf_v)
    pltpu.sync_copy(dstf_r.at[c, s], df_v)
    pltpu.sync_copy(zc_r, cacc_v)

    def cbody(k, carry):
        sidx = sf_v[pl.ds(k * 16, 16)]
        didx = df_v[pl.ds(k * 16, 16)]
        vals = plsc.load_gather(nrm_v, [didx])
        plsc.addupdate_scatter(cacc_v, [sidx], vals)
        return carry

    lax.fori_loop(0, VCH, cbody, 0)
    pltpu.sync_copy(cacc_v, cp_o.at[c, s])


def _p5_body(alo, ahi, glo, ghi, n_ref, cp, w1, b1, w2, b2, out):
    rb = pl.program_id(1)
    ncol = n_ref[0, 0]
    x_lo = (alo[0, 0] + glo[0, 0]) * ncol
    x_hi = (ahi[0, 0] + ghi[0, 0]) * ncol
    z = (jnp.dot(x_lo, w1[0, :H, :], preferred_element_type=f32)
         + jnp.dot(x_hi, w1[0, H:, :], preferred_element_type=f32)
         + b1[0])
    h1 = jnp.where(z >= 0, z, 0.01 * z)
    csum = jnp.sum(cp[0, 0], axis=1, keepdims=True)
    row = rb * RB + lax.broadcasted_iota(i32, (RB, 1), 0)
    w = jnp.where(row < N, ncol * (csum + ncol) * (1.0 / N), 0.0)
    sp = lax.dot_general(w, h1, (((0,), (0,)), ((), ())),
                         preferred_element_type=f32)

    @pl.when(rb == 0)
    def _():
        out[...] = jnp.zeros_like(out)

    out[0] += sp

    @pl.when(rb == pl.num_programs(1) - 1)
    def _():
        out[0] = (jnp.dot(out[0], w2[0], preferred_element_type=f32)
                  + b2[0])


_p5 = pl.pallas_call(
    _p5_body,
    grid=(2, NRB),
    in_specs=[
        pl.BlockSpec((1, 1, RB, H), lambda g, b: (0, g, b, 0)),
        pl.BlockSpec((1, 1, RB, H), lambda g, b: (1, g, b, 0)),
        pl.BlockSpec((1, 1, RB, H), lambda g, b: (0, g, b, 0)),
        pl.BlockSpec((1, 1, RB, H), lambda g, b: (1, g, b, 0)),
        pl.BlockSpec((1, 1, RB, 1), lambda g, b: (g, b, 0, 0)),
        pl.BlockSpec((1, 1, RB, NS), lambda g, b: (g, b, 0, 0)),
        pl.BlockSpec((1, D, D), lambda g, b: (g, 0, 0)),
        pl.BlockSpec((1, 1, D), lambda g, b: (g, 0, 0)),
        pl.BlockSpec((1, D, D), lambda g, b: (g, 0, 0)),
        pl.BlockSpec((1, 1, D), lambda g, b: (g, 0, 0)),
    ],
    out_specs=pl.BlockSpec((1, 1, D), lambda g, b: (g, 0, 0)),
    out_shape=jax.ShapeDtypeStruct((2, 1, D), f32),
)


def kernel(ids_u, pos_u, edge_index_u, ids_v, pos_v, edge_index_v,
           emb, Wp1, bp1, Wp2, bp2, Wc1, bc1, Wc2, bc2):
    del pos_u, pos_v
    su = edge_index_u[0].astype(i32)
    du = edge_index_u[1].astype(i32)
    sv = edge_index_v[0].astype(i32)
    dv = edge_index_v[1].astype(i32)

    pad = jnp.full((2, NS, EPTP - EPT), NP - 1, i32)
    srcp = jnp.concatenate(
        [jnp.stack([su, sv]).reshape(2, NS, EPT), pad], axis=2)
    dstp = jnp.concatenate(
        [jnp.stack([du, dv]).reshape(2, NS, EPT), pad], axis=2)
    dst_r = dstp
    hoff = (jnp.arange(2, dtype=i32) * (2 * NP))[:, None, None, None]
    goff = (jnp.arange(2, dtype=i32) * NP)[None, :, None, None]
    srcA = (srcp[None] + hoff + goff).reshape(NC, 2, NS, NBLK, NBC, K)
    dst3 = dstp.reshape(2, NS, NBLK, NBC, K)
    srcf = srcp

    ids2 = jnp.stack([ids_u, ids_v]).astype(i32)
    ids_pad = jnp.concatenate(
        [ids2, jnp.zeros((2, NP - N), i32)], axis=1).reshape(2, NS, GNIT, GK)

    zc = jnp.zeros((NP,), f32)
    zrows = jnp.zeros((K, H), f32)

    h0, degp = _p1(ids_pad, dst_r, emb, zc)
    norm2 = _p2a(degp)
    norm_c = norm2.reshape(NC, NRB, RB, 1)
    gout = _p2b(h0, norm_c)
    g2 = gout.reshape(2 * NC * NP, H)
    agg = _p3(srcA, dst3, g2, zrows)
    cpart = _p3c(srcf, dst_r, norm2, zc)
    cpt = cpart.reshape(2, NS, NRB, RB).transpose(0, 2, 3, 1)

    W1s = jnp.stack([Wp1, Wc1])
    b1s = jnp.stack([bp1, bc1]).reshape(2, 1, D)
    W2s = jnp.stack([Wp2, Wc2])
    b2s = jnp.stack([bp2, bc2]).reshape(2, 1, D)

    outp = _p5(agg, agg, gout, gout, norm_c, cpt, W1s, b1s, W2s, b2s)
    return outp.reshape(2, D)


# speedup vs baseline: 1.8030x; 1.8030x over previous
import functools

import jax
import jax.numpy as jnp
from jax import lax
from jax.experimental import pallas as pl
from jax.experimental.pallas import tpu as pltpu
from jax.experimental.pallas import tpu_sc as plsc

N = 10000
E = 160000
D = 256
H = 128
NC, NS = 2, 16
NP = 10240
RPT = NP // NS
GK = 128
GNIT = RPT // GK
EPT = E // NS
K = 80
NIT = EPT // K
VCH = EPT // 16
RB = 1024
NRB = NP // RB

f32 = jnp.float32
i32 = jnp.int32

_mesh = plsc.VectorSubcoreMesh(core_axis_name="c", subcore_axis_name="s")


@functools.partial(
    pl.kernel,
    out_type=(
        jax.ShapeDtypeStruct((NC, NP, D), f32),
        jax.ShapeDtypeStruct((NC, NS, NP), f32),
    ),
    mesh=_mesh,
    compiler_params=pltpu.CompilerParams(needs_layout_passes=False),
    scratch_types=(
        pltpu.VMEM((GNIT, GK), i32),
        pltpu.VMEM((GK, D), f32),
        pltpu.VMEM((GK, D), f32),
        pltpu.VMEM((EPT,), i32),
        pltpu.VMEM((NP,), f32),
        pltpu.SemaphoreType.DMA,
        pltpu.SemaphoreType.DMA,
    ),
)
def _p1(ids_r, dst_r, emb_r, zc_r, h0_o, deg_o, ids_v, r0_v, r1_v, dst_v,
        acc_v, semA, semB):
    c = lax.axis_index("c")
    s = lax.axis_index("s")
    bufs = (r0_v, r1_v)
    sems = (semA, semB)
    pltpu.sync_copy(ids_r.at[c, s], ids_v)
    pltpu.async_copy(emb_r.at[ids_v.at[0]], r0_v, semA)
    pltpu.async_copy(emb_r.at[ids_v.at[1]], r1_v, semB)
    pltpu.sync_copy(dst_r.at[c, s], dst_v)
    pltpu.sync_copy(zc_r, acc_v)
    ones = jnp.full((16,), 1.0, f32)

    def deg_body(k, carry):
        idx = dst_v[pl.ds(k * 16, 16)]
        plsc.addupdate_scatter(acc_v, [idx], ones)
        return carry

    lax.fori_loop(0, VCH, deg_body, 0)
    pltpu.sync_copy(acc_v, deg_o.at[c, s])
    dummy = emb_r.at[pl.ds(0, GK), :]
    for j in range(GNIT):
        b, sm = bufs[j % 2], sems[j % 2]
        pltpu.make_async_copy(dummy, b, sm).wait()
        if j + 2 < GNIT:
            nb_, nsm = bufs[j % 2], sems[j % 2]
            pltpu.sync_copy(b, h0_o.at[c, pl.ds(s * RPT + j * GK, GK), :])
            pltpu.async_copy(emb_r.at[ids_v.at[j + 2]], nb_, nsm)
        else:
            pltpu.sync_copy(b, h0_o.at[c, pl.ds(s * RPT + j * GK, GK), :])


def _p2b_body(h0_ref, dp_ref, g_ref, n_ref):
    deg = jnp.sum(dp_ref[0, 0], axis=1, keepdims=True) + 1.0
    ncol = lax.rsqrt(deg)
    gf = h0_ref[0] * ncol
    g_ref[0, 0] = gf[:, :H]
    g_ref[1, 0] = gf[:, H:]
    n_ref[0, 0] = ncol


_p2b = pl.pallas_call(
    _p2b_body,
    grid=(NC, NRB),
    in_specs=[
        pl.BlockSpec((1, RB, D), lambda g, b: (g, b, 0)),
        pl.BlockSpec((1, 1, RB, NS), lambda g, b: (g, b, 0, 0)),
    ],
    out_specs=[
        pl.BlockSpec((2, 1, RB, H), lambda g, b: (0, g, b, 0)),
        pl.BlockSpec((1, 1, RB, 1), lambda g, b: (g, b, 0, 0)),
    ],
    out_shape=[
        jax.ShapeDtypeStruct((2, NC, NP, H), f32),
        jax.ShapeDtypeStruct((NC, NRB, RB, 1), f32),
    ],
)


@functools.partial(
    pl.kernel,
    out_type=jax.ShapeDtypeStruct((NC, 2, NP, H), f32),
    mesh=_mesh,
    compiler_params=pltpu.CompilerParams(needs_layout_passes=False),
    scratch_types=(
        pltpu.VMEM((NIT // 5, K), i32),
        pltpu.VMEM((NIT // 5, K), i32),
        pltpu.VMEM((K, H), f32),
        pltpu.VMEM((K, H), f32),
        pltpu.VMEM((K, H), f32),
        pltpu.VMEM_SHARED((NP, H), f32),
        pltpu.SemaphoreType.DMA,
        pltpu.SemaphoreType.DMA,
        pltpu.SemaphoreType.DMA,
        pltpu.SemaphoreType.DMA,
        pltpu.SemaphoreType.DMA,
        pltpu.SemaphoreType.DMA,
    ),
)
def _p3(srcA_r, dst3_r, g2_r, zrows_r, agg_o, si_v, di_v, r0_v, r1_v, r2_v,
        acc_sh, sg0, sg1, sg2, ss0, ss1, ss2):
    c = lax.axis_index("c")
    s = lax.axis_index("s")
    B = (r0_v, r1_v, r2_v)
    SG = (sg0, sg1, sg2)
    SS = (ss0, ss1, ss2)
    dummy = g2_r.at[pl.ds(0, K), :]

    def wG(b):
        pltpu.make_async_copy(dummy, B[b], SG[b]).wait()

    def wS(b):
        pltpu.make_async_copy(dummy, B[b], SS[b]).wait()

    for g in range(2):
        pltpu.sync_copy(zrows_r, r0_v)
        for j in range(RPT // K):
            pltpu.sync_copy(r0_v, acc_sh.at[pl.ds(s * RPT + j * K, K), :])
        plsc.subcore_barrier()

        for blk in range(5):
            pltpu.sync_copy(srcA_r.at[c, g, s, blk], si_v)
            pltpu.sync_copy(dst3_r.at[g, s, blk], di_v)

            def iG(b, j):
                pltpu.async_copy(g2_r.at[si_v.at[j]], B[b], SG[b])

            def iS(b, j):
                pltpu.async_copy(B[b], acc_sh.at[di_v.at[j]], SS[b],
                                 add=True)

            iG(0, 0)
            iG(1, 1)
            wG(0); iS(0, 0); iG(2, 2)
            wG(1); iS(1, 1); wS(0); iG(0, 3)

            def body(ii, carry):
                j0 = 2 + ii * 3
                for t, (b, bp) in enumerate(((2, 1), (0, 2), (1, 0))):
                    j = j0 + t
                    wG(b); iS(b, j); wS(bp); iG(bp, j + 2)
                return carry

            lax.fori_loop(0, 7, body, 0)
            wG(2); iS(2, 23)
            wG(0); iS(0, 24)
            wS(1); wS(2); wS(0)

        plsc.subcore_barrier()
        for j in range(RPT // K):
            r0 = s * RPT + j * K
            pltpu.sync_copy(acc_sh.at[pl.ds(r0, K), :], r0_v)
            pltpu.sync_copy(r0_v, agg_o.at[c, g, pl.ds(r0, K), :])
        plsc.subcore_barrier()


@functools.partial(
    pl.kernel,
    out_type=jax.ShapeDtypeStruct((NC, NS, NP), f32),
    mesh=_mesh,
    compiler_params=pltpu.CompilerParams(needs_layout_passes=False),
    scratch_types=(
        pltpu.VMEM((EPT,), i32),
        pltpu.VMEM((EPT,), i32),
        pltpu.VMEM((NP,), f32),
        pltpu.VMEM((NP,), f32),
    ),
)
def _p3c(srcf_r, dstf_r, norm_r, zc_r, cp_o, sf_v, df_v, nrm_v, cacc_v):
    c = lax.axis_index("c")
    s = lax.axis_index("s")
    pltpu.sync_copy(norm_r.at[c], nrm_v)
    pltpu.sync_copy(srcf_r.at[c, s], sf_v)
    pltpu.sync_copy(dstf_r.at[c, s], df_v)
    pltpu.sync_copy(zc_r, cacc_v)

    def cbody(k, carry):
        sidx = sf_v[pl.ds(k * 16, 16)]
        didx = df_v[pl.ds(k * 16, 16)]
        vals = plsc.load_gather(nrm_v, [didx])
        plsc.addupdate_scatter(cacc_v, [sidx], vals)
        return carry

    lax.fori_loop(0, VCH, cbody, 0)
    pltpu.sync_copy(cacc_v, cp_o.at[c, s])


def _p5_body(alo, ahi, glo, ghi, n_ref, cp, w1, b1, w2, b2, out):
    rb = pl.program_id(1)
    ncol = n_ref[0, 0]
    x_lo = (alo[0, 0] + glo[0, 0]) * ncol
    x_hi = (ahi[0, 0] + ghi[0, 0]) * ncol
    z = (jnp.dot(x_lo, w1[0, :H, :], preferred_element_type=f32)
         + jnp.dot(x_hi, w1[0, H:, :], preferred_element_type=f32)
         + b1[0])
    h1 = jnp.where(z >= 0, z, 0.01 * z)
    csum = jnp.sum(cp[0, 0], axis=1, keepdims=True)
    row = rb * RB + lax.broadcasted_iota(i32, (RB, 1), 0)
    w = jnp.where(row < N, ncol * (csum + ncol) * (1.0 / N), 0.0)
    sp = lax.dot_general(w, h1, (((0,), (0,)), ((), ())),
                         preferred_element_type=f32)

    @pl.when(rb == 0)
    def _():
        out[...] = jnp.zeros_like(out)

    out[0] += sp

    @pl.when(rb == pl.num_programs(1) - 1)
    def _():
        out[0] = (jnp.dot(out[0], w2[0], preferred_element_type=f32)
                  + b2[0])


_p5 = pl.pallas_call(
    _p5_body,
    grid=(2, NRB),
    in_specs=[
        pl.BlockSpec((1, 1, RB, H), lambda g, b: (0, g, b, 0)),
        pl.BlockSpec((1, 1, RB, H), lambda g, b: (1, g, b, 0)),
        pl.BlockSpec((1, 1, RB, H), lambda g, b: (0, g, b, 0)),
        pl.BlockSpec((1, 1, RB, H), lambda g, b: (1, g, b, 0)),
        pl.BlockSpec((1, 1, RB, 1), lambda g, b: (g, b, 0, 0)),
        pl.BlockSpec((1, 1, RB, NS), lambda g, b: (g, b, 0, 0)),
        pl.BlockSpec((1, D, D), lambda g, b: (g, 0, 0)),
        pl.BlockSpec((1, 1, D), lambda g, b: (g, 0, 0)),
        pl.BlockSpec((1, D, D), lambda g, b: (g, 0, 0)),
        pl.BlockSpec((1, 1, D), lambda g, b: (g, 0, 0)),
    ],
    out_specs=pl.BlockSpec((1, 1, D), lambda g, b: (g, 0, 0)),
    out_shape=jax.ShapeDtypeStruct((2, 1, D), f32),
)


def kernel(ids_u, pos_u, edge_index_u, ids_v, pos_v, edge_index_v,
           emb, Wp1, bp1, Wp2, bp2, Wc1, bc1, Wc2, bc2):
    del pos_u, pos_v
    su = edge_index_u[0].astype(i32)
    du = edge_index_u[1].astype(i32)
    sv = edge_index_v[0].astype(i32)
    dv = edge_index_v[1].astype(i32)

    dst_r = jnp.stack([du, dv]).reshape(2, NS, EPT)
    src2 = jnp.stack([su, sv])
    hoff = (jnp.arange(2, dtype=i32) * (2 * NP))[:, None, None]
    goff = (jnp.arange(2, dtype=i32) * NP)[None, :, None]
    srcA = (src2[None] + hoff + goff).reshape(NC, 2, NS, 5, NIT // 5, K)
    dst3 = jnp.stack([du, dv]).reshape(2, NS, 5, NIT // 5, K)
    srcf = src2.reshape(2, NS, EPT)

    ids2 = jnp.stack([ids_u, ids_v]).astype(i32)
    ids_pad = jnp.concatenate(
        [ids2, jnp.zeros((2, NP - N), i32)], axis=1).reshape(2, NS, GNIT, GK)

    zc = jnp.zeros((NP,), f32)
    zrows = jnp.zeros((K, H), f32)

    h0, degp = _p1(ids_pad, dst_r, emb, zc)
    dpt = degp.reshape(NC, NS, NRB, RB).transpose(0, 2, 3, 1)
    gout, norm_c = _p2b(h0, dpt)
    norm2 = norm_c.reshape(NC, NP)
    g2 = gout.reshape(2 * NC * NP, H)
    agg = _p3(srcA, dst3, g2, zrows)
    cpart = _p3c(srcf, dst_r, norm2, zc)
    cpt = cpart.reshape(2, NS, NRB, RB).transpose(0, 2, 3, 1)

    W1s = jnp.stack([Wp1, Wc1])
    b1s = jnp.stack([bp1, bc1]).reshape(2, 1, D)
    W2s = jnp.stack([Wp2, Wc2])
    b2s = jnp.stack([bp2, bc2]).reshape(2, 1, D)

    outp = _p5(agg, agg, gout, gout, norm_c, cpt, W1s, b1s, W2s, b2s)
    return outp.reshape(2, D)


# speedup vs baseline: 1.8933x; 1.0501x over previous
import functools

import jax
import jax.numpy as jnp
from jax import lax
from jax.experimental import pallas as pl
from jax.experimental.pallas import tpu as pltpu
from jax.experimental.pallas import tpu_sc as plsc

N = 10000
E = 160000
D = 256
H = 128
NC, NS = 2, 16
NP = 10240
RPT = NP // NS
GK = 128
GNIT = RPT // GK
EPT = E // NS
K = 80
NIT = EPT // K
VCH = EPT // 16
RB = 1024
NRB = NP // RB

f32 = jnp.float32
i32 = jnp.int32

_mesh = plsc.VectorSubcoreMesh(core_axis_name="c", subcore_axis_name="s")


@functools.partial(
    pl.kernel,
    out_type=(
        jax.ShapeDtypeStruct((NC, NP, D), f32),
        jax.ShapeDtypeStruct((NC, NS, NP), f32),
    ),
    mesh=_mesh,
    compiler_params=pltpu.CompilerParams(needs_layout_passes=False),
    scratch_types=(
        pltpu.VMEM((GNIT, GK), i32),
        pltpu.VMEM((GK, D), f32),
        pltpu.VMEM((GK, D), f32),
        pltpu.VMEM((EPT,), i32),
        pltpu.VMEM((NP,), f32),
        pltpu.SemaphoreType.DMA,
        pltpu.SemaphoreType.DMA,
    ),
)
def _p1(ids_r, dst_r, emb_r, zc_r, h0_o, deg_o, ids_v, r0_v, r1_v, dst_v,
        acc_v, semA, semB):
    c = lax.axis_index("c")
    s = lax.axis_index("s")
    bufs = (r0_v, r1_v)
    sems = (semA, semB)
    pltpu.sync_copy(ids_r.at[c, s], ids_v)
    pltpu.async_copy(emb_r.at[ids_v.at[0]], r0_v, semA)
    pltpu.async_copy(emb_r.at[ids_v.at[1]], r1_v, semB)
    pltpu.sync_copy(dst_r.at[c, s], dst_v)
    pltpu.sync_copy(zc_r, acc_v)
    ones = jnp.full((16,), 1.0, f32)

    def deg_body(k, carry):
        idx = dst_v[pl.ds(k * 16, 16)]
        plsc.addupdate_scatter(acc_v, [idx], ones)
        return carry

    lax.fori_loop(0, VCH, deg_body, 0)
    pltpu.sync_copy(acc_v, deg_o.at[c, s])
    dummy = emb_r.at[pl.ds(0, GK), :]
    for j in range(GNIT):
        b, sm = bufs[j % 2], sems[j % 2]
        pltpu.make_async_copy(dummy, b, sm).wait()
        if j + 2 < GNIT:
            nb_, nsm = bufs[j % 2], sems[j % 2]
            pltpu.sync_copy(b, h0_o.at[c, pl.ds(s * RPT + j * GK, GK), :])
            pltpu.async_copy(emb_r.at[ids_v.at[j + 2]], nb_, nsm)
        else:
            pltpu.sync_copy(b, h0_o.at[c, pl.ds(s * RPT + j * GK, GK), :])


def _p2a_body(dp_ref, norm_ref):
    d = jnp.sum(dp_ref[...], axis=1) + 1.0
    norm_ref[...] = lax.rsqrt(d)


_p2a = pl.pallas_call(
    _p2a_body,
    out_shape=jax.ShapeDtypeStruct((NC, NP), f32),
)


def _p2b_body(h0_ref, n_ref, g_ref):
    ncol = n_ref[0, 0]
    gf = h0_ref[0] * ncol
    g_ref[0, 0] = gf[:, :H]
    g_ref[1, 0] = gf[:, H:]


_p2b = pl.pallas_call(
    _p2b_body,
    grid=(NC, NRB),
    in_specs=[
        pl.BlockSpec((1, RB, D), lambda g, b: (g, b, 0)),
        pl.BlockSpec((1, 1, RB, 1), lambda g, b: (g, b, 0, 0)),
    ],
    out_specs=pl.BlockSpec((2, 1, RB, H), lambda g, b: (0, g, b, 0)),
    out_shape=jax.ShapeDtypeStruct((2, NC, NP, H), f32),
)


@functools.partial(
    pl.kernel,
    out_type=jax.ShapeDtypeStruct((NC, 2, NP, H), f32),
    mesh=_mesh,
    compiler_params=pltpu.CompilerParams(needs_layout_passes=False),
    scratch_types=(
        pltpu.VMEM((NIT // 5, K), i32),
        pltpu.VMEM((NIT // 5, K), i32),
        pltpu.VMEM((K, H), f32),
        pltpu.VMEM((K, H), f32),
        pltpu.VMEM((K, H), f32),
        pltpu.VMEM_SHARED((NP, H), f32),
        pltpu.SemaphoreType.DMA,
        pltpu.SemaphoreType.DMA,
        pltpu.SemaphoreType.DMA,
        pltpu.SemaphoreType.DMA,
        pltpu.SemaphoreType.DMA,
        pltpu.SemaphoreType.DMA,
    ),
)
def _p3(srcA_r, dst3_r, g2_r, zrows_r, agg_o, si_v, di_v, r0_v, r1_v, r2_v,
        acc_sh, sg0, sg1, sg2, ss0, ss1, ss2):
    c = lax.axis_index("c")
    s = lax.axis_index("s")
    B = (r0_v, r1_v, r2_v)
    SG = (sg0, sg1, sg2)
    SS = (ss0, ss1, ss2)
    dummy = g2_r.at[pl.ds(0, K), :]

    def wG(b):
        pltpu.make_async_copy(dummy, B[b], SG[b]).wait()

    def wS(b):
        pltpu.make_async_copy(dummy, B[b], SS[b]).wait()

    for g in range(2):
        pltpu.sync_copy(zrows_r, r0_v)
        for j in range(RPT // K):
            pltpu.sync_copy(r0_v, acc_sh.at[pl.ds(s * RPT + j * K, K), :])
        plsc.subcore_barrier()

        for blk in range(5):
            pltpu.sync_copy(srcA_r.at[c, g, s, blk], si_v)
            pltpu.sync_copy(dst3_r.at[g, s, blk], di_v)

            def iG(b, j):
                pltpu.async_copy(g2_r.at[si_v.at[j]], B[b], SG[b])

            def iS(b, j):
                pltpu.async_copy(B[b], acc_sh.at[di_v.at[j]], SS[b],
                                 add=True)

            iG(0, 0)
            iG(1, 1)
            wG(0); iS(0, 0); iG(2, 2)
            wS(0); iG(0, 3); wG(1); iS(1, 1)

            def body(ii, carry):
                j0 = 2 + ii * 3
                for t, (b, bp) in enumerate(((2, 1), (0, 2), (1, 0))):
                    j = j0 + t
                    wS(bp); iG(bp, j + 2); wG(b); iS(b, j)
                return carry

            lax.fori_loop(0, 7, body, 0)
            wS(1); wG(2); iS(2, 23)
            wG(0); iS(0, 24)
            wS(2); wS(0)

        plsc.subcore_barrier()
        for j in range(RPT // K):
            r0 = s * RPT + j * K
            pltpu.sync_copy(acc_sh.at[pl.ds(r0, K), :], r0_v)
            pltpu.sync_copy(r0_v, agg_o.at[c, g, pl.ds(r0, K), :])
        plsc.subcore_barrier()


@functools.partial(
    pl.kernel,
    out_type=jax.ShapeDtypeStruct((NC, NS, NP), f32),
    mesh=_mesh,
    compiler_params=pltpu.CompilerParams(needs_layout_passes=False),
    scratch_types=(
        pltpu.VMEM((EPT,), i32),
        pltpu.VMEM((EPT,), i32),
        pltpu.VMEM((NP,), f32),
        pltpu.VMEM((NP,), f32),
    ),
)
def _p3c(srcf_r, dstf_r, norm_r, zc_r, cp_o, sf_v, df_v, nrm_v, cacc_v):
    c = lax.axis_index("c")
    s = lax.axis_index("s")
    pltpu.sync_copy(norm_r.at[c], nrm_v)
    pltpu.sync_copy(srcf_r.at[c, s], sf_v)
    pltpu.sync_copy(dstf_r.at[c, s], df_v)
    pltpu.sync_copy(zc_r, cacc_v)

    def cbody(k, carry):
        sidx = sf_v[pl.ds(k * 16, 16)]
        didx = df_v[pl.ds(k * 16, 16)]
        vals = plsc.load_gather(nrm_v, [didx])
        plsc.addupdate_scatter(cacc_v, [sidx], vals)
        return carry

    lax.fori_loop(0, VCH, cbody, 0)
    pltpu.sync_copy(cacc_v, cp_o.at[c, s])


def _p5_body(alo, ahi, glo, ghi, n_ref, cp, w1, b1, w2, b2, out):
    rb = pl.program_id(1)
    ncol = n_ref[0, 0]
    x_lo = (alo[0, 0] + glo[0, 0]) * ncol
    x_hi = (ahi[0, 0] + ghi[0, 0]) * ncol
    z = (jnp.dot(x_lo, w1[0, :H, :], preferred_element_type=f32)
         + jnp.dot(x_hi, w1[0, H:, :], preferred_element_type=f32)
         + b1[0])
    h1 = jnp.where(z >= 0, z, 0.01 * z)
    csum = jnp.sum(cp[0, 0], axis=1, keepdims=True)
    row = rb * RB + lax.broadcasted_iota(i32, (RB, 1), 0)
    w = jnp.where(row < N, ncol * (csum + ncol) * (1.0 / N), 0.0)
    sp = lax.dot_general(w, h1, (((0,), (0,)), ((), ())),
                         preferred_element_type=f32)

    @pl.when(rb == 0)
    def _():
        out[...] = jnp.zeros_like(out)

    out[0] += sp

    @pl.when(rb == pl.num_programs(1) - 1)
    def _():
        out[0] = (jnp.dot(out[0], w2[0], preferred_element_type=f32)
                  + b2[0])


_p5 = pl.pallas_call(
    _p5_body,
    grid=(2, NRB),
    in_specs=[
        pl.BlockSpec((1, 1, RB, H), lambda g, b: (0, g, b, 0)),
        pl.BlockSpec((1, 1, RB, H), lambda g, b: (1, g, b, 0)),
        pl.BlockSpec((1, 1, RB, H), lambda g, b: (0, g, b, 0)),
        pl.BlockSpec((1, 1, RB, H), lambda g, b: (1, g, b, 0)),
        pl.BlockSpec((1, 1, RB, 1), lambda g, b: (g, b, 0, 0)),
        pl.BlockSpec((1, 1, RB, NS), lambda g, b: (g, b, 0, 0)),
        pl.BlockSpec((1, D, D), lambda g, b: (g, 0, 0)),
        pl.BlockSpec((1, 1, D), lambda g, b: (g, 0, 0)),
        pl.BlockSpec((1, D, D), lambda g, b: (g, 0, 0)),
        pl.BlockSpec((1, 1, D), lambda g, b: (g, 0, 0)),
    ],
    out_specs=pl.BlockSpec((1, 1, D), lambda g, b: (g, 0, 0)),
    out_shape=jax.ShapeDtypeStruct((2, 1, D), f32),
)


def kernel(ids_u, pos_u, edge_index_u, ids_v, pos_v, edge_index_v,
           emb, Wp1, bp1, Wp2, bp2, Wc1, bc1, Wc2, bc2):
    del pos_u, pos_v
    su = edge_index_u[0].astype(i32)
    du = edge_index_u[1].astype(i32)
    sv = edge_index_v[0].astype(i32)
    dv = edge_index_v[1].astype(i32)

    dst_r = jnp.stack([du, dv]).reshape(2, NS, EPT)
    src2 = jnp.stack([su, sv])
    hoff = (jnp.arange(2, dtype=i32) * (2 * NP))[:, None, None]
    goff = (jnp.arange(2, dtype=i32) * NP)[None, :, None]
    srcA = (src2[None] + hoff + goff).reshape(NC, 2, NS, 5, NIT // 5, K)
    dst3 = jnp.stack([du, dv]).reshape(2, NS, 5, NIT // 5, K)
    srcf = src2.reshape(2, NS, EPT)

    ids2 = jnp.stack([ids_u, ids_v]).astype(i32)
    ids_pad = jnp.concatenate(
        [ids2, jnp.zeros((2, NP - N), i32)], axis=1).reshape(2, NS, GNIT, GK)

    zc = jnp.zeros((NP,), f32)
    zrows = jnp.zeros((K, H), f32)

    h0, degp = _p1(ids_pad, dst_r, emb, zc)
    norm2 = _p2a(degp)
    norm_c = norm2.reshape(NC, NRB, RB, 1)
    gout = _p2b(h0, norm_c)
    g2 = gout.reshape(2 * NC * NP, H)
    agg = _p3(srcA, dst3, g2, zrows)
    cpart = _p3c(srcf, dst_r, norm2, zc)
    cpt = cpart.reshape(2, NS, NRB, RB).transpose(0, 2, 3, 1)

    W1s = jnp.stack([Wp1, Wc1])
    b1s = jnp.stack([bp1, bc1]).reshape(2, 1, D)
    W2s = jnp.stack([Wp2, Wc2])
    b2s = jnp.stack([bp2, bc2]).reshape(2, 1, D)

    outp = _p5(agg, agg, gout, gout, norm_c, cpt, W1s, b1s, W2s, b2s)
    return outp.reshape(2, D)
